# baseline (speedup 1.0000x reference)
import functools

import jax
import jax.numpy as jnp
from jax.experimental import pallas as pl
from jax.experimental.pallas import tpu as pltpu

_BN_EPS = 1e-5


def _bn_act(y, aff_ref):
    m = jnp.mean(y, axis=0, keepdims=True)
    v = jnp.mean((y - m) ** 2, axis=0, keepdims=True)
    return jnp.maximum(
        (y - m) * (aff_ref[1:2, :] * jax.lax.rsqrt(v + _BN_EPS)) + aff_ref[2:3, :], 0.0)


def _se_kernel(w1t_ref, a1_ref, w2_ref, a2_ref, x_ref, o_ref,
               xs_ref, pool_ref, gate_ref, *, ni, tin, no, tout, chunk, inv_hw):
    i = pl.program_id(0)
    r = tin // tout
    stash_rows = (ni - 1) * tin

    @pl.when(i < ni)
    def _pool_and_stash():
        xv = x_ref[...].astype(jnp.float32)
        s = jnp.sum(xv, axis=0)

        @pl.when(i == 0)
        def _():
            pool_ref[...] = s

        @pl.when(i > 0)
        def _():
            pool_ref[...] += s

        @pl.when(i < ni - 1)
        def _():
            for b in range(0, tin, chunk):
                xs_ref[pl.ds(i * tin + b, chunk)] = xv[b:b + chunk]

    @pl.when(i == ni)
    def _excite():
        a = pool_ref[...] * inv_hw
        y1 = jax.lax.dot_general(a, w1t_ref[...], (((1,), (1,)), ((), ())),
                                 preferred_element_type=jnp.float32) + a1_ref[0:1, :]
        h1 = _bn_act(y1, a1_ref)
        y2 = jax.lax.dot_general(h1, w2_ref[...], (((1,), (0,)), ((), ())),
                                 preferred_element_type=jnp.float32) + a2_ref[0:1, :]
        h2 = _bn_act(y2, a2_ref)
        gate_ref[...] = 1.0 / (1.0 + jnp.exp(-h2))

    @pl.when(i >= ni)
    def _scale():
        j = i - ni
        g = gate_ref[...]

        @pl.when(j < (ni - 1) * r)
        def _():
            o_ref[...] = (xs_ref[pl.ds(j * tout, tout)].astype(jnp.float32)
                          * g[None, :, :]).astype(o_ref.dtype)

        @pl.when(j >= (ni - 1) * r)
        def _():
            row = j * tout - stash_rows
            o_ref[...] = (x_ref[pl.ds(row, tout)].astype(jnp.float32)
                          * g[None, :, :]).astype(o_ref.dtype)


def kernel(x, w1t, w2t, aff1, aff2):
    n, c, h, w = x.shape
    hw = h * w
    cr = w1t.shape[0]
    xt = x.transpose(2, 3, 0, 1).reshape(hw, n, c)

    tin = hw
    for cand in (196, 112, 98, 64, 56, 49, 28, 16, 8, 7, 4, 2, 1):
        if hw % cand == 0:
            tin = cand
            break
    ni = hw // tin
    tout = tin
    no = hw // tout
    chunk = tin
    while chunk * n * c > 384 * 8 * 128 and chunk % 2 == 0:
        chunk //= 2

    body = functools.partial(_se_kernel, ni=ni, tin=tin, no=no, tout=tout,
                             chunk=chunk, inv_hw=1.0 / float(hw))
    out = pl.pallas_call(
        body,
        out_shape=jax.ShapeDtypeStruct((hw, n, c), x.dtype),
        grid=(ni + no,),
        in_specs=[
            pl.BlockSpec((cr, c), lambda i: (0, 0)),
            pl.BlockSpec((3, cr), lambda i: (0, 0)),
            pl.BlockSpec((cr, c), lambda i: (0, 0)),
            pl.BlockSpec((3, c), lambda i: (0, 0)),
            pl.BlockSpec((tin, n, c), lambda i: (jnp.minimum(i, ni - 1), 0, 0)),
        ],
        out_specs=pl.BlockSpec((tout, n, c), lambda i: (jnp.maximum(i - ni, 0), 0, 0)),
        scratch_shapes=[
            pltpu.VMEM(((hw // tin - 1) * tin, n, c), jnp.float32),
            pltpu.VMEM((n, c), jnp.float32),
            pltpu.VMEM((n, c), jnp.float32),
        ],
        compiler_params=pltpu.CompilerParams(
            dimension_semantics=("arbitrary",),
            vmem_limit_bytes=50 * 1024 * 1024),
        name="se_fused",
    )(w1t, aff1.T, w2t.T, aff2.T, xt)
    return out.reshape(h, w, n, c).transpose(2, 3, 0, 1)


# speedup vs baseline: 1.1360x; 1.1360x over previous
import functools

import jax
import jax.numpy as jnp
from jax.experimental import pallas as pl
from jax.experimental.pallas import tpu as pltpu

_BN_EPS = 1e-5


def _bn_act(y, aff_ref):
    m = jnp.mean(y, axis=0, keepdims=True)
    v = jnp.mean((y - m) ** 2, axis=0, keepdims=True)
    return jnp.maximum(
        (y - m) * (aff_ref[1:2, :] * jax.lax.rsqrt(v + _BN_EPS)) + aff_ref[2:3, :], 0.0)


def _se_kernel(w1t_ref, a1_ref, w2t_ref, a2_ref, x_ref, o_ref,
               xs_ref, pool_ref, gate_ref, *, nt, thw, chunk, inv_hw):
    i = pl.program_id(0)

    @pl.when(i < nt)
    def _pool_and_stash():
        xv = x_ref[...].astype(jnp.float32)
        s = jnp.sum(xv, axis=0)

        @pl.when(i == 0)
        def _():
            pool_ref[...] = s

        @pl.when(i > 0)
        def _():
            pool_ref[...] += s

        for b in range(0, thw, chunk):
            xs_ref[pl.ds(i * thw + b, chunk)] = xv[b:b + chunk]

    @pl.when(i == nt)
    def _excite():
        a = pool_ref[...] * inv_hw
        y1 = jax.lax.dot_general(a, w1t_ref[...], (((1,), (1,)), ((), ())),
                                 preferred_element_type=jnp.float32) + a1_ref[0:1, :]
        h1 = _bn_act(y1, a1_ref)
        y2 = jax.lax.dot_general(h1, w2t_ref[...], (((1,), (0,)), ((), ())),
                                 preferred_element_type=jnp.float32) + a2_ref[0:1, :]
        h2 = _bn_act(y2, a2_ref)
        gate_ref[...] = 1.0 / (1.0 + jnp.exp(-h2))

    @pl.when(i >= nt)
    def _scale():
        j = i - nt
        g = gate_ref[...]
        o_ref[...] = (xs_ref[pl.ds(j * thw, thw)] * g[None, :, :]).astype(o_ref.dtype)


def kernel(x, w1t, w2t, aff1, aff2):
    n, c, h, w = x.shape
    hw = h * w
    cr = w1t.shape[0]
    xt = x.transpose(2, 3, 0, 1).reshape(hw, n, c)

    thw = hw
    for cand in (196, 112, 98, 64, 56, 49, 28, 16, 8, 7, 4, 2, 1):
        if hw % cand == 0:
            thw = cand
            break
    nt = hw // thw
    chunk = thw
    while chunk * n * c > 384 * 8 * 128 and chunk % 2 == 0:
        chunk //= 2

    body = functools.partial(_se_kernel, nt=nt, thw=thw, chunk=chunk,
                             inv_hw=1.0 / float(hw))
    out = pl.pallas_call(
        body,
        out_shape=jax.ShapeDtypeStruct((hw, n, c), x.dtype),
        grid=(2 * nt,),
        in_specs=[
            pl.BlockSpec((cr, c), lambda i: (0, 0)),
            pl.BlockSpec((3, cr), lambda i: (0, 0)),
            pl.BlockSpec((cr, c), lambda i: (0, 0)),
            pl.BlockSpec((3, c), lambda i: (0, 0)),
            pl.BlockSpec((thw, n, c), lambda i: (jnp.minimum(i, nt - 1), 0, 0)),
        ],
        out_specs=pl.BlockSpec((thw, n, c), lambda i: (jnp.maximum(i - nt, 0), 0, 0)),
        scratch_shapes=[
            pltpu.VMEM((hw, n, c), jnp.float32),
            pltpu.VMEM((n, c), jnp.float32),
            pltpu.VMEM((n, c), jnp.float32),
        ],
        compiler_params=pltpu.CompilerParams(
            dimension_semantics=("arbitrary",),
            vmem_limit_bytes=57 * 1024 * 1024),
        name="se_fused",
    )(w1t, aff1.T, w2t.T, aff2.T, xt)
    return out.reshape(h, w, n, c).transpose(2, 3, 0, 1)
